# Initial kernel scaffold; baseline (speedup 1.0000x reference)
import jax
import jax.numpy as jnp
from jax.experimental import pallas as pl


def kernel(input, W2, W3):
    raise NotImplementedError("write your pallas kernel here")


# speedup vs baseline: 117.9817x; 117.9817x over previous
import jax
import jax.numpy as jnp
from jax.experimental import pallas as pl

BLOCK_B = 512


def _body(x_ref, w2_ref, w3_ref, o_ref):
    x = x_ref[...].astype(jnp.int32)

    v = x - ((x >> 1) & 0x55555555)
    v = (v & 0x33333333) + ((v >> 2) & 0x33333333)
    v = (v + (v >> 4)) & 0x0F0F0F0F
    p = ((v + (v >> 8)) & 0x1F).astype(jnp.float32)

    tf = x.astype(jnp.float32)
    third = jnp.float32(0.33333334)
    c1 = jnp.zeros_like(tf)
    c2 = jnp.zeros_like(tf)
    for _ in range(10):
        q = jnp.floor((tf + 0.5) * third)
        d = tf - 3.0 * q
        c1 = c1 + (d == 1.0).astype(jnp.float32)
        c2 = c2 + (d == 2.0).astype(jnp.float32)
        tf = q

    w2 = w2_ref[...]
    w3 = w3_ref[...]
    base2 = jnp.reshape(16.0 * w2[0:1, :], (1, 1, 64))
    dw2 = jnp.reshape(w2[1:2, :] - w2[0:1, :], (1, 1, 64))
    base3 = jnp.reshape(16.0 * w3[0:1, :], (1, 1, 64))
    dw31 = jnp.reshape(w3[1:2, :] - w3[0:1, :], (1, 1, 64))
    dw32 = jnp.reshape(w3[2:3, :] - w3[0:1, :], (1, 1, 64))

    p3 = p[:, :, None]
    c13 = c1[:, :, None]
    c23 = c2[:, :, None]
    o_ref[:, :, 0:64] = base2 + p3 * dw2
    o_ref[:, :, 64:128] = base3 + c13 * dw31 + c23 * dw32


def kernel(input, W2, W3):
    x = input.astype(jnp.int32)
    B, F = x.shape
    D = W2.shape[1]
    grid = (B // BLOCK_B,)
    out = pl.pallas_call(
        _body,
        grid=grid,
        in_specs=[
            pl.BlockSpec((BLOCK_B, F), lambda i: (i, 0)),
            pl.BlockSpec((2, D), lambda i: (0, 0)),
            pl.BlockSpec((3, D), lambda i: (0, 0)),
        ],
        out_specs=pl.BlockSpec((BLOCK_B, F, 2 * D), lambda i: (i, 0, 0)),
        out_shape=jax.ShapeDtypeStruct((B, F, 2 * D), jnp.float32),
    )(x, W2, W3)
    return out
